# baseline (speedup 1.0000x reference)
import jax
import jax.numpy as jnp
from jax import lax
from jax.experimental import pallas as pl
from jax.experimental.pallas import tpu as pltpu

_NUM_EXPERTS = 64
_TOP_K = 8
_HIDDEN = 2048
_NUM_TOKENS = 16384
_BLK = 1024
_NSTREAM = 2


def _router_body(x0_ref, x1_ref, wt_ref, w0_ref, i0_ref, w1_ref, i1_ref,
                 aux_ref, cnt_ref, psum_ref):
    i = pl.program_id(0)
    nblocks = pl.num_programs(0)

    @pl.when(i == 0)
    def _init():
        cnt_ref[...] = jnp.zeros_like(cnt_ref)
        psum_ref[...] = jnp.zeros_like(psum_ref)
        aux_ref[...] = jnp.zeros((1, 1), jnp.float32)

    eidx = lax.broadcasted_iota(jnp.int32, (_NUM_EXPERTS, _BLK), 0)

    for x_ref, wout_ref, iout_ref in ((x0_ref, w0_ref, i0_ref),
                                      (x1_ref, w1_ref, i1_ref)):
        logits = lax.dot_general(
            wt_ref[...], x_ref[...],
            dimension_numbers=(((1,), (1,)), ((), ())),
            preferred_element_type=jnp.float32)
        m = jnp.max(logits, axis=0, keepdims=True)
        e = jnp.exp(logits - m)
        s = jnp.sum(e, axis=0, keepdims=True)

        cur = e
        sel = jnp.zeros((_NUM_EXPERTS, _BLK), jnp.float32)
        vals = []
        idxs = []
        for _ in range(_TOP_K):
            mx = jnp.max(cur, axis=0, keepdims=True)
            hit = cur == mx
            amx = jnp.min(jnp.where(hit, eidx, _NUM_EXPERTS),
                          axis=0, keepdims=True)
            pick = eidx == amx
            vals.append(mx)
            idxs.append(amx)
            sel = jnp.where(pick, 1.0, sel)
            cur = jnp.where(pick, -1.0, cur)

        w8 = jnp.concatenate(vals, axis=0)
        i8 = jnp.concatenate(idxs, axis=0)
        wout_ref[...] = w8 / jnp.sum(w8, axis=0, keepdims=True)
        iout_ref[...] = i8

        probs = e * (1.0 / s)
        cnt_ref[...] += jnp.sum(sel, axis=1, keepdims=True)
        psum_ref[...] += jnp.sum(probs, axis=1, keepdims=True)

    @pl.when(i == nblocks - 1)
    def _fin():
        f = cnt_ref[...] / (_NUM_TOKENS * _TOP_K)
        p = psum_ref[...] / _NUM_TOKENS
        aux_ref[...] = _NUM_EXPERTS * jnp.sum(f * p, keepdims=True).reshape(1, 1)


def kernel(hidden_states, gate_w):
    nblocks = _NUM_TOKENS // (_BLK * _NSTREAM)
    wt = gate_w.T
    w0, i0, w1, i1, aux = pl.pallas_call(
        _router_body,
        grid=(nblocks,),
        in_specs=[
            pl.BlockSpec((_BLK, _HIDDEN), lambda i: (2 * i, 0)),
            pl.BlockSpec((_BLK, _HIDDEN), lambda i: (2 * i + 1, 0)),
            pl.BlockSpec((_NUM_EXPERTS, _HIDDEN), lambda i: (0, 0)),
        ],
        out_specs=[
            pl.BlockSpec((_TOP_K, _BLK), lambda i: (0, i)),
            pl.BlockSpec((_TOP_K, _BLK), lambda i: (0, i)),
            pl.BlockSpec((_TOP_K, _BLK), lambda i: (0, i)),
            pl.BlockSpec((_TOP_K, _BLK), lambda i: (0, i)),
            pl.BlockSpec((1, 1), lambda i: (0, 0)),
        ],
        out_shape=[
            jax.ShapeDtypeStruct((_TOP_K, _NUM_TOKENS // 2), jnp.float32),
            jax.ShapeDtypeStruct((_TOP_K, _NUM_TOKENS // 2), jnp.int32),
            jax.ShapeDtypeStruct((_TOP_K, _NUM_TOKENS // 2), jnp.float32),
            jax.ShapeDtypeStruct((_TOP_K, _NUM_TOKENS // 2), jnp.int32),
            jax.ShapeDtypeStruct((1, 1), jnp.float32),
        ],
        scratch_shapes=[
            pltpu.VMEM((_NUM_EXPERTS, 1), jnp.float32),
            pltpu.VMEM((_NUM_EXPERTS, 1), jnp.float32),
        ],
    )(hidden_states, hidden_states, wt)

    def _merge(a, b):
        a = a.reshape(_TOP_K, nblocks, 1, _BLK)
        b = b.reshape(_TOP_K, nblocks, 1, _BLK)
        return (jnp.concatenate([a, b], axis=2)
                .reshape(_TOP_K, _NUM_TOKENS).T)

    return (_merge(w0, w1), _merge(i0, i1), aux[0, 0])
